# Initial kernel scaffold; baseline (speedup 1.0000x reference)
import jax
import jax.numpy as jnp
from jax.experimental import pallas as pl


def kernel(x_cat, x_cont, tables, gc, bc, W1, b1, g1, bt1, W2, b2, g2, bt2, W3, b3):
    raise NotImplementedError("write your pallas kernel here")


# speedup vs baseline: 4.5523x; 4.5523x over previous
import functools

import jax
import jax.numpy as jnp
from jax import lax
from jax.experimental import pallas as pl
from jax.experimental.pallas import tpu as pltpu
from jax.experimental.pallas import tpu_sc as plsc

B = 16384
F = 26
V = 100000
D = 16
NC = 13
H1 = 512
H2 = 256
FD = F * D
EPS = 1e-5

_NW = 32
_TOT = B * F
_PW = _TOT // _NW
_IDX_ROWS = _PW // 128
_GROUP = 1024
_G_STEPS = _PW // _GROUP

_BT = 1024
_T = B // _BT


def _sc_gather(flat_tables, idx2d):
    mesh = plsc.VectorSubcoreMesh(core_axis_name="c", subcore_axis_name="s")

    @functools.partial(
        pl.kernel,
        mesh=mesh,
        out_type=jax.ShapeDtypeStruct((_TOT, D), jnp.float32),
        scratch_types=[
            pltpu.VMEM((_IDX_ROWS, 128), jnp.int32),
            pltpu.VMEM((_GROUP, D), jnp.float32),
            pltpu.SemaphoreType.DMA,
        ],
        compiler_params=pltpu.CompilerParams(use_tc_tiling_on_sc=False),
    )
    def k(table_hbm, idx_hbm, out_hbm, idx_v, rows_v, sem):
        wid = lax.axis_index("s") * 2 + lax.axis_index("c")
        row0 = wid * _IDX_ROWS
        pltpu.sync_copy(idx_hbm.at[pl.ds(row0, _IDX_ROWS)], idx_v)

        def body(g, carry):
            cps = []
            for j in range(_GROUP // 128):
                cps.append(pltpu.async_copy(
                    table_hbm.at[idx_v.at[g * (_GROUP // 128) + j]],
                    rows_v.at[pl.ds(j * 128, 128)],
                    sem))
            for cp in cps:
                cp.wait()
            pltpu.sync_copy(
                rows_v, out_hbm.at[pl.ds(wid * _PW + g * _GROUP, _GROUP)])
            return carry

        lax.fori_loop(0, _G_STEPS, body, 0)

    return k(flat_tables, idx2d)


def _stage1(emb, xc, gc, bc, W1e, W1c, b1):
    def body(emb_ref, xc_ref, gc_ref, bc_ref, w1e_ref, w1c_ref, b1_ref,
             h_ref, s_ref, ss_ref, xcn_ref):
        t = pl.program_id(0)

        @pl.when(t == 0)
        def _():
            x = xc_ref[...]
            m = jnp.mean(x, axis=0, keepdims=True)
            v = jnp.mean((x - m) ** 2, axis=0, keepdims=True)
            xcn_ref[...] = (gc_ref[...] * (x - m) / jnp.sqrt(v + EPS)
                            + bc_ref[...])
            s_ref[...] = jnp.zeros_like(s_ref)
            ss_ref[...] = jnp.zeros_like(ss_ref)

        xcn = xcn_ref[pl.ds(t * _BT, _BT), :]
        h = emb_ref[...] @ w1e_ref[...] + xcn @ w1c_ref[...] + b1_ref[...]
        h = jnp.maximum(h, 0.0)
        h_ref[...] = h
        s_ref[...] += jnp.sum(h, axis=0, keepdims=True)
        ss_ref[...] += jnp.sum(h * h, axis=0, keepdims=True)

    return pl.pallas_call(
        body,
        grid=(_T,),
        in_specs=[
            pl.BlockSpec((_BT, FD), lambda t: (t, 0)),
            pl.BlockSpec((B, NC), lambda t: (0, 0)),
            pl.BlockSpec((1, NC), lambda t: (0, 0)),
            pl.BlockSpec((1, NC), lambda t: (0, 0)),
            pl.BlockSpec((FD, H1), lambda t: (0, 0)),
            pl.BlockSpec((NC, H1), lambda t: (0, 0)),
            pl.BlockSpec((1, H1), lambda t: (0, 0)),
        ],
        out_specs=[
            pl.BlockSpec((_BT, H1), lambda t: (t, 0)),
            pl.BlockSpec((1, H1), lambda t: (0, 0)),
            pl.BlockSpec((1, H1), lambda t: (0, 0)),
        ],
        out_shape=[
            jax.ShapeDtypeStruct((B, H1), jnp.float32),
            jax.ShapeDtypeStruct((1, H1), jnp.float32),
            jax.ShapeDtypeStruct((1, H1), jnp.float32),
        ],
        scratch_shapes=[pltpu.VMEM((B, NC), jnp.float32)],
        compiler_params=pltpu.CompilerParams(
            dimension_semantics=("arbitrary",)),
    )(emb, xc, gc, bc, W1e, W1c, b1)


def _stage2(h1, s1, ss1, g1, bt1, W2, b2):
    def body(h_ref, s_ref, ss_ref, g_ref, bt_ref, w2_ref, b2_ref,
             h2_ref, s2_ref, ss2_ref):
        t = pl.program_id(0)
        m = s_ref[...] * (1.0 / B)
        var = ss_ref[...] * (1.0 / B) - m * m
        scale = g_ref[...] * lax.rsqrt(var + EPS)
        shift = bt_ref[...] - m * scale
        z = h_ref[...] * scale + shift
        h2 = jnp.maximum(z @ w2_ref[...] + b2_ref[...], 0.0)
        h2_ref[...] = h2

        @pl.when(t == 0)
        def _():
            s2_ref[...] = jnp.zeros_like(s2_ref)
            ss2_ref[...] = jnp.zeros_like(ss2_ref)

        s2_ref[...] += jnp.sum(h2, axis=0, keepdims=True)
        ss2_ref[...] += jnp.sum(h2 * h2, axis=0, keepdims=True)

    return pl.pallas_call(
        body,
        grid=(_T,),
        in_specs=[
            pl.BlockSpec((_BT, H1), lambda t: (t, 0)),
            pl.BlockSpec((1, H1), lambda t: (0, 0)),
            pl.BlockSpec((1, H1), lambda t: (0, 0)),
            pl.BlockSpec((1, H1), lambda t: (0, 0)),
            pl.BlockSpec((1, H1), lambda t: (0, 0)),
            pl.BlockSpec((H1, H2), lambda t: (0, 0)),
            pl.BlockSpec((1, H2), lambda t: (0, 0)),
        ],
        out_specs=[
            pl.BlockSpec((_BT, H2), lambda t: (t, 0)),
            pl.BlockSpec((1, H2), lambda t: (0, 0)),
            pl.BlockSpec((1, H2), lambda t: (0, 0)),
        ],
        out_shape=[
            jax.ShapeDtypeStruct((B, H2), jnp.float32),
            jax.ShapeDtypeStruct((1, H2), jnp.float32),
            jax.ShapeDtypeStruct((1, H2), jnp.float32),
        ],
        compiler_params=pltpu.CompilerParams(
            dimension_semantics=("arbitrary",)),
    )(h1, s1, ss1, g1, bt1, W2, b2)


def _stage3(h2, s2, ss2, g2, bt2, W3, b3):
    def body(h_ref, s_ref, ss_ref, g_ref, bt_ref, w3_ref, b3_ref, o_ref):
        m = s_ref[...] * (1.0 / B)
        var = ss_ref[...] * (1.0 / B) - m * m
        scale = g_ref[...] * lax.rsqrt(var + EPS)
        shift = bt_ref[...] - m * scale
        z = h_ref[...] * scale + shift
        o_ref[...] = z @ w3_ref[...] + b3_ref[...]

    return pl.pallas_call(
        body,
        grid=(_T,),
        in_specs=[
            pl.BlockSpec((_BT, H2), lambda t: (t, 0)),
            pl.BlockSpec((1, H2), lambda t: (0, 0)),
            pl.BlockSpec((1, H2), lambda t: (0, 0)),
            pl.BlockSpec((1, H2), lambda t: (0, 0)),
            pl.BlockSpec((1, H2), lambda t: (0, 0)),
            pl.BlockSpec((H2, 1), lambda t: (0, 0)),
            pl.BlockSpec((1, 1), lambda t: (0, 0)),
        ],
        out_specs=pl.BlockSpec((_BT, 1), lambda t: (t, 0)),
        out_shape=jax.ShapeDtypeStruct((B, 1), jnp.float32),
        compiler_params=pltpu.CompilerParams(
            dimension_semantics=("arbitrary",)),
    )(h2, s2, ss2, g2, bt2, W3, b3)


def kernel(x_cat, x_cont, tables, gc, bc, W1, b1, g1, bt1, W2, b2, g2, bt2,
           W3, b3):
    flat_tables = tables.reshape(F * V, D)
    offsets = (jnp.arange(F) * V).astype(jnp.int32)
    flat_idx = (x_cat.astype(jnp.int32) + offsets[None, :]).reshape(-1)
    idx2d = flat_idx.reshape(_TOT // 128, 128)

    emb = _sc_gather(flat_tables, idx2d).reshape(B, FD)

    W1e = W1[:FD, :]
    W1c = W1[FD:, :]
    h1, s1, ss1 = _stage1(emb, x_cont, gc.reshape(1, NC), bc.reshape(1, NC),
                          W1e, W1c, b1.reshape(1, H1))
    h2, s2, ss2 = _stage2(h1, s1, ss1, g1.reshape(1, H1), bt1.reshape(1, H1),
                          W2, b2.reshape(1, H2))
    out = _stage3(h2, s2, ss2, g2.reshape(1, H2), bt2.reshape(1, H2),
                  W3, b3.reshape(1, 1))
    return out


# speedup vs baseline: 17.8759x; 3.9268x over previous
import functools

import jax
import jax.numpy as jnp
from jax import lax
from jax.experimental import pallas as pl
from jax.experimental.pallas import tpu as pltpu
from jax.experimental.pallas import tpu_sc as plsc

B = 16384
F = 26
V = 100000
D = 16
NC = 13
H1 = 512
H2 = 256
FD = F * D
EPS = 1e-5

_NW = 32
_TOT = B * F
_PW = _TOT // _NW
_IDX_ROWS = _PW // 128
_GROUP = 1024
_G_STEPS = _PW // _GROUP

_BT = 1024
_T = B // _BT


_VP = 100352
_FR = _VP * D // 128
_NC = 2
_BR = _FR // _NC


def _tc_transpose(mv):
    def body(*refs):
        in_refs, out_ref = refs[:8], refs[8]
        x = jnp.concatenate([r[...] for r in in_refs], axis=0)
        out_ref[...] = x.T

    def make_map(j):
        return lambda f, c: (f, _NC * j + c)

    return pl.pallas_call(
        body,
        grid=(F, _NC),
        in_specs=[pl.BlockSpec((D, _BR), make_map(j)) for j in range(8)],
        out_specs=pl.BlockSpec((_BR, 128), lambda f, c: (_NC * f + c, 0)),
        out_shape=jax.ShapeDtypeStruct((F * _FR, 128), jnp.float32),
    )(*([mv] * 8))


def _sc_gather(flat_tables, idx2d):
    mesh = plsc.VectorSubcoreMesh(core_axis_name="c", subcore_axis_name="s")

    @functools.partial(
        pl.kernel,
        mesh=mesh,
        out_type=jax.ShapeDtypeStruct((_TOT, D), jnp.float32),
        scratch_types=[
            pltpu.VMEM((_IDX_ROWS, 128), jnp.int32),
            pltpu.VMEM((_GROUP, D), jnp.float32),
            pltpu.SemaphoreType.DMA,
        ],
        compiler_params=pltpu.CompilerParams(use_tc_tiling_on_sc=False),
    )
    def k(table_hbm, idx_hbm, out_hbm, idx_v, rows_v, sem):
        wid = lax.axis_index("s") * 2 + lax.axis_index("c")
        row0 = wid * _IDX_ROWS
        pltpu.sync_copy(idx_hbm.at[pl.ds(row0, _IDX_ROWS)], idx_v)

        def body(g, carry):
            cps = []
            for j in range(_GROUP // 128):
                cps.append(pltpu.async_copy(
                    table_hbm.at[idx_v.at[g * (_GROUP // 128) + j]],
                    rows_v.at[pl.ds(j * 128, 128)],
                    sem))
            for cp in cps:
                cp.wait()
            pltpu.sync_copy(
                rows_v, out_hbm.at[pl.ds(wid * _PW + g * _GROUP, _GROUP)])
            return carry

        lax.fori_loop(0, _G_STEPS, body, 0)

    return k(flat_tables, idx2d)


def _stage1(emb, xc, gc, bc, W1e, W1c, b1):
    def body(emb_ref, xc_ref, gc_ref, bc_ref, w1e_ref, w1c_ref, b1_ref,
             h_ref, s_ref, ss_ref, xcn_ref):
        t = pl.program_id(0)

        @pl.when(t == 0)
        def _():
            x = xc_ref[...]
            m = jnp.mean(x, axis=0, keepdims=True)
            v = jnp.mean((x - m) ** 2, axis=0, keepdims=True)
            xcn_ref[...] = (gc_ref[...] * (x - m) / jnp.sqrt(v + EPS)
                            + bc_ref[...])
            s_ref[...] = jnp.zeros_like(s_ref)
            ss_ref[...] = jnp.zeros_like(ss_ref)

        xcn = xcn_ref[pl.ds(t * _BT, _BT), :]
        h = emb_ref[...] @ w1e_ref[...] + xcn @ w1c_ref[...] + b1_ref[...]
        h = jnp.maximum(h, 0.0)
        h_ref[...] = h
        s_ref[...] += jnp.sum(h, axis=0, keepdims=True)
        ss_ref[...] += jnp.sum(h * h, axis=0, keepdims=True)

    return pl.pallas_call(
        body,
        grid=(_T,),
        in_specs=[
            pl.BlockSpec((_BT, FD), lambda t: (t, 0)),
            pl.BlockSpec((B, NC), lambda t: (0, 0)),
            pl.BlockSpec((1, NC), lambda t: (0, 0)),
            pl.BlockSpec((1, NC), lambda t: (0, 0)),
            pl.BlockSpec((FD, H1), lambda t: (0, 0)),
            pl.BlockSpec((NC, H1), lambda t: (0, 0)),
            pl.BlockSpec((1, H1), lambda t: (0, 0)),
        ],
        out_specs=[
            pl.BlockSpec((_BT, H1), lambda t: (t, 0)),
            pl.BlockSpec((1, H1), lambda t: (0, 0)),
            pl.BlockSpec((1, H1), lambda t: (0, 0)),
        ],
        out_shape=[
            jax.ShapeDtypeStruct((B, H1), jnp.float32),
            jax.ShapeDtypeStruct((1, H1), jnp.float32),
            jax.ShapeDtypeStruct((1, H1), jnp.float32),
        ],
        scratch_shapes=[pltpu.VMEM((B, NC), jnp.float32)],
        compiler_params=pltpu.CompilerParams(
            dimension_semantics=("arbitrary",)),
    )(emb, xc, gc, bc, W1e, W1c, b1)


def _stage2(h1, s1, ss1, g1, bt1, W2, b2):
    def body(h_ref, s_ref, ss_ref, g_ref, bt_ref, w2_ref, b2_ref,
             h2_ref, s2_ref, ss2_ref):
        t = pl.program_id(0)
        m = s_ref[...] * (1.0 / B)
        var = ss_ref[...] * (1.0 / B) - m * m
        scale = g_ref[...] * lax.rsqrt(var + EPS)
        shift = bt_ref[...] - m * scale
        z = h_ref[...] * scale + shift
        h2 = jnp.maximum(z @ w2_ref[...] + b2_ref[...], 0.0)
        h2_ref[...] = h2

        @pl.when(t == 0)
        def _():
            s2_ref[...] = jnp.zeros_like(s2_ref)
            ss2_ref[...] = jnp.zeros_like(ss2_ref)

        s2_ref[...] += jnp.sum(h2, axis=0, keepdims=True)
        ss2_ref[...] += jnp.sum(h2 * h2, axis=0, keepdims=True)

    return pl.pallas_call(
        body,
        grid=(_T,),
        in_specs=[
            pl.BlockSpec((_BT, H1), lambda t: (t, 0)),
            pl.BlockSpec((1, H1), lambda t: (0, 0)),
            pl.BlockSpec((1, H1), lambda t: (0, 0)),
            pl.BlockSpec((1, H1), lambda t: (0, 0)),
            pl.BlockSpec((1, H1), lambda t: (0, 0)),
            pl.BlockSpec((H1, H2), lambda t: (0, 0)),
            pl.BlockSpec((1, H2), lambda t: (0, 0)),
        ],
        out_specs=[
            pl.BlockSpec((_BT, H2), lambda t: (t, 0)),
            pl.BlockSpec((1, H2), lambda t: (0, 0)),
            pl.BlockSpec((1, H2), lambda t: (0, 0)),
        ],
        out_shape=[
            jax.ShapeDtypeStruct((B, H2), jnp.float32),
            jax.ShapeDtypeStruct((1, H2), jnp.float32),
            jax.ShapeDtypeStruct((1, H2), jnp.float32),
        ],
        compiler_params=pltpu.CompilerParams(
            dimension_semantics=("arbitrary",)),
    )(h1, s1, ss1, g1, bt1, W2, b2)


def _stage3(h2, s2, ss2, g2, bt2, W3, b3):
    def body(h_ref, s_ref, ss_ref, g_ref, bt_ref, w3_ref, b3_ref, o_ref):
        m = s_ref[...] * (1.0 / B)
        var = ss_ref[...] * (1.0 / B) - m * m
        scale = g_ref[...] * lax.rsqrt(var + EPS)
        shift = bt_ref[...] - m * scale
        z = h_ref[...] * scale + shift
        o_ref[...] = z @ w3_ref[...] + b3_ref[...]

    return pl.pallas_call(
        body,
        grid=(_T,),
        in_specs=[
            pl.BlockSpec((_BT, H2), lambda t: (t, 0)),
            pl.BlockSpec((1, H2), lambda t: (0, 0)),
            pl.BlockSpec((1, H2), lambda t: (0, 0)),
            pl.BlockSpec((1, H2), lambda t: (0, 0)),
            pl.BlockSpec((1, H2), lambda t: (0, 0)),
            pl.BlockSpec((H2, 1), lambda t: (0, 0)),
            pl.BlockSpec((1, 1), lambda t: (0, 0)),
        ],
        out_specs=pl.BlockSpec((_BT, 1), lambda t: (t, 0)),
        out_shape=jax.ShapeDtypeStruct((B, 1), jnp.float32),
        compiler_params=pltpu.CompilerParams(
            dimension_semantics=("arbitrary",)),
    )(h2, s2, ss2, g2, bt2, W3, b3)


def kernel(x_cat, x_cont, tables, gc, bc, W1, b1, g1, bt1, W2, b2, g2, bt2,
           W3, b3):
    mv = jnp.transpose(tables, (0, 2, 1)).reshape(F * D, V)
    flat_tables = _tc_transpose(mv).reshape(F * _VP, D)
    offsets = (jnp.arange(F) * _VP).astype(jnp.int32)
    v = x_cat.astype(jnp.int32)
    perm = (v % _FR) * 8 + v // _FR
    flat_idx = (perm + offsets[None, :]).reshape(-1)
    idx2d = flat_idx.reshape(_TOT // 128, 128)

    emb = _sc_gather(flat_tables, idx2d).reshape(B, FD)

    W1e = W1[:FD, :]
    W1c = W1[FD:, :]
    h1, s1, ss1 = _stage1(emb, x_cont, gc.reshape(1, NC), bc.reshape(1, NC),
                          W1e, W1c, b1.reshape(1, H1))
    h2, s2, ss2 = _stage2(h1, s1, ss1, g1.reshape(1, H1), bt1.reshape(1, H1),
                          W2, b2.reshape(1, H2))
    out = _stage3(h2, s2, ss2, g2.reshape(1, H2), bt2.reshape(1, H2),
                  W3, b3.reshape(1, 1))
    return out


# speedup vs baseline: 18.3795x; 1.0282x over previous
import functools

import jax
import jax.numpy as jnp
from jax import lax
from jax.experimental import pallas as pl
from jax.experimental.pallas import tpu as pltpu
from jax.experimental.pallas import tpu_sc as plsc

B = 16384
F = 26
V = 100000
D = 16
NC = 13
H1 = 512
H2 = 256
FD = F * D
EPS = 1e-5

_NW = 32
_TOT = B * F
_PW = _TOT // _NW
_IDX_ROWS = _PW // 128
_GROUP = 1024
_G_STEPS = _PW // _GROUP

_BT = 1024
_T = B // _BT


_VP = 100352
_FR = _VP * D // 128
_NC = 1
_BR = _FR // _NC


def _tc_transpose(mv):
    def body(*refs):
        in_refs, out_ref = refs[:8], refs[8]
        x = jnp.concatenate([r[...] for r in in_refs], axis=0)
        out_ref[...] = x.T

    def make_map(j):
        return lambda f, c: (f, _NC * j + c)

    return pl.pallas_call(
        body,
        grid=(F, _NC),
        in_specs=[pl.BlockSpec((D, _BR), make_map(j)) for j in range(8)],
        out_specs=pl.BlockSpec((_BR, 128), lambda f, c: (_NC * f + c, 0)),
        out_shape=jax.ShapeDtypeStruct((F * _FR, 128), jnp.float32),
    )(*([mv] * 8))


def _sc_gather(flat_tables, idx2d):
    mesh = plsc.VectorSubcoreMesh(core_axis_name="c", subcore_axis_name="s")

    @functools.partial(
        pl.kernel,
        mesh=mesh,
        out_type=jax.ShapeDtypeStruct((_TOT, D), jnp.float32),
        scratch_types=[
            pltpu.VMEM((_IDX_ROWS, 128), jnp.int32),
            pltpu.VMEM((_GROUP, D), jnp.float32),
            pltpu.SemaphoreType.DMA,
        ],
        compiler_params=pltpu.CompilerParams(use_tc_tiling_on_sc=False),
    )
    def k(table_hbm, idx_hbm, out_hbm, idx_v, rows_v, sem):
        wid = lax.axis_index("s") * 2 + lax.axis_index("c")
        row0 = wid * _IDX_ROWS
        pltpu.sync_copy(idx_hbm.at[pl.ds(row0, _IDX_ROWS)], idx_v)

        def body(g, carry):
            cps = []
            for j in range(_GROUP // 128):
                cps.append(pltpu.async_copy(
                    table_hbm.at[idx_v.at[g * (_GROUP // 128) + j]],
                    rows_v.at[pl.ds(j * 128, 128)],
                    sem))
            for cp in cps:
                cp.wait()
            pltpu.sync_copy(
                rows_v, out_hbm.at[pl.ds(wid * _PW + g * _GROUP, _GROUP)])
            return carry

        lax.fori_loop(0, _G_STEPS, body, 0)

    return k(flat_tables, idx2d)


def _stage1(emb, xc, gc, bc, W1e, W1c, b1):
    def body(emb_ref, xc_ref, gc_ref, bc_ref, w1e_ref, w1c_ref, b1_ref,
             h_ref, s_ref, ss_ref, xcn_ref):
        t = pl.program_id(0)

        @pl.when(t == 0)
        def _():
            x = xc_ref[...]
            m = jnp.mean(x, axis=0, keepdims=True)
            v = jnp.mean((x - m) ** 2, axis=0, keepdims=True)
            xcn_ref[...] = (gc_ref[...] * (x - m) / jnp.sqrt(v + EPS)
                            + bc_ref[...])
            s_ref[...] = jnp.zeros_like(s_ref)
            ss_ref[...] = jnp.zeros_like(ss_ref)

        xcn = xcn_ref[pl.ds(t * _BT, _BT), :]
        h = emb_ref[...] @ w1e_ref[...] + xcn @ w1c_ref[...] + b1_ref[...]
        h = jnp.maximum(h, 0.0)
        h_ref[...] = h
        s_ref[...] += jnp.sum(h, axis=0, keepdims=True)
        ss_ref[...] += jnp.sum(h * h, axis=0, keepdims=True)

    return pl.pallas_call(
        body,
        grid=(_T,),
        in_specs=[
            pl.BlockSpec((_BT, FD), lambda t: (t, 0)),
            pl.BlockSpec((B, NC), lambda t: (0, 0)),
            pl.BlockSpec((1, NC), lambda t: (0, 0)),
            pl.BlockSpec((1, NC), lambda t: (0, 0)),
            pl.BlockSpec((FD, H1), lambda t: (0, 0)),
            pl.BlockSpec((NC, H1), lambda t: (0, 0)),
            pl.BlockSpec((1, H1), lambda t: (0, 0)),
        ],
        out_specs=[
            pl.BlockSpec((_BT, H1), lambda t: (t, 0)),
            pl.BlockSpec((1, H1), lambda t: (0, 0)),
            pl.BlockSpec((1, H1), lambda t: (0, 0)),
        ],
        out_shape=[
            jax.ShapeDtypeStruct((B, H1), jnp.float32),
            jax.ShapeDtypeStruct((1, H1), jnp.float32),
            jax.ShapeDtypeStruct((1, H1), jnp.float32),
        ],
        scratch_shapes=[pltpu.VMEM((B, NC), jnp.float32)],
        compiler_params=pltpu.CompilerParams(
            dimension_semantics=("arbitrary",)),
    )(emb, xc, gc, bc, W1e, W1c, b1)


def _stage2(h1, s1, ss1, g1, bt1, W2, b2):
    def body(h_ref, s_ref, ss_ref, g_ref, bt_ref, w2_ref, b2_ref,
             h2_ref, s2_ref, ss2_ref):
        t = pl.program_id(0)
        m = s_ref[...] * (1.0 / B)
        var = ss_ref[...] * (1.0 / B) - m * m
        scale = g_ref[...] * lax.rsqrt(var + EPS)
        shift = bt_ref[...] - m * scale
        z = h_ref[...] * scale + shift
        h2 = jnp.maximum(z @ w2_ref[...] + b2_ref[...], 0.0)
        h2_ref[...] = h2

        @pl.when(t == 0)
        def _():
            s2_ref[...] = jnp.zeros_like(s2_ref)
            ss2_ref[...] = jnp.zeros_like(ss2_ref)

        s2_ref[...] += jnp.sum(h2, axis=0, keepdims=True)
        ss2_ref[...] += jnp.sum(h2 * h2, axis=0, keepdims=True)

    return pl.pallas_call(
        body,
        grid=(_T,),
        in_specs=[
            pl.BlockSpec((_BT, H1), lambda t: (t, 0)),
            pl.BlockSpec((1, H1), lambda t: (0, 0)),
            pl.BlockSpec((1, H1), lambda t: (0, 0)),
            pl.BlockSpec((1, H1), lambda t: (0, 0)),
            pl.BlockSpec((1, H1), lambda t: (0, 0)),
            pl.BlockSpec((H1, H2), lambda t: (0, 0)),
            pl.BlockSpec((1, H2), lambda t: (0, 0)),
        ],
        out_specs=[
            pl.BlockSpec((_BT, H2), lambda t: (t, 0)),
            pl.BlockSpec((1, H2), lambda t: (0, 0)),
            pl.BlockSpec((1, H2), lambda t: (0, 0)),
        ],
        out_shape=[
            jax.ShapeDtypeStruct((B, H2), jnp.float32),
            jax.ShapeDtypeStruct((1, H2), jnp.float32),
            jax.ShapeDtypeStruct((1, H2), jnp.float32),
        ],
        compiler_params=pltpu.CompilerParams(
            dimension_semantics=("arbitrary",)),
    )(h1, s1, ss1, g1, bt1, W2, b2)


def _stage3(h2, s2, ss2, g2, bt2, W3, b3):
    def body(h_ref, s_ref, ss_ref, g_ref, bt_ref, w3_ref, b3_ref, o_ref):
        m = s_ref[...] * (1.0 / B)
        var = ss_ref[...] * (1.0 / B) - m * m
        scale = g_ref[...] * lax.rsqrt(var + EPS)
        shift = bt_ref[...] - m * scale
        z = h_ref[...] * scale + shift
        o_ref[...] = z @ w3_ref[...] + b3_ref[...]

    return pl.pallas_call(
        body,
        grid=(_T,),
        in_specs=[
            pl.BlockSpec((_BT, H2), lambda t: (t, 0)),
            pl.BlockSpec((1, H2), lambda t: (0, 0)),
            pl.BlockSpec((1, H2), lambda t: (0, 0)),
            pl.BlockSpec((1, H2), lambda t: (0, 0)),
            pl.BlockSpec((1, H2), lambda t: (0, 0)),
            pl.BlockSpec((H2, 1), lambda t: (0, 0)),
            pl.BlockSpec((1, 1), lambda t: (0, 0)),
        ],
        out_specs=pl.BlockSpec((_BT, 1), lambda t: (t, 0)),
        out_shape=jax.ShapeDtypeStruct((B, 1), jnp.float32),
        compiler_params=pltpu.CompilerParams(
            dimension_semantics=("arbitrary",)),
    )(h2, s2, ss2, g2, bt2, W3, b3)


def kernel(x_cat, x_cont, tables, gc, bc, W1, b1, g1, bt1, W2, b2, g2, bt2,
           W3, b3):
    mv = jnp.transpose(tables, (0, 2, 1)).reshape(F * D, V)
    flat_tables = _tc_transpose(mv).reshape(F * _VP, D)
    offsets = (jnp.arange(F) * _VP).astype(jnp.int32)
    v = x_cat.astype(jnp.int32)
    perm = (v % _FR) * 8 + v // _FR
    flat_idx = (perm + offsets[None, :]).reshape(-1)
    idx2d = flat_idx.reshape(_TOT // 128, 128)

    emb = _sc_gather(flat_tables, idx2d).reshape(B, FD)

    W1e = W1[:FD, :]
    W1c = W1[FD:, :]
    h1, s1, ss1 = _stage1(emb, x_cont, gc.reshape(1, NC), bc.reshape(1, NC),
                          W1e, W1c, b1.reshape(1, H1))
    h2, s2, ss2 = _stage2(h1, s1, ss1, g1.reshape(1, H1), bt1.reshape(1, H1),
                          W2, b2.reshape(1, H2))
    out = _stage3(h2, s2, ss2, g2.reshape(1, H2), bt2.reshape(1, H2),
                  W3, b3.reshape(1, 1))
    return out
